# baseline (speedup 1.0000x reference)
import functools

import jax
import jax.numpy as jnp
from jax import lax
from jax.experimental import pallas as pl
from jax.experimental.pallas import tpu as pltpu
from jax.experimental.pallas import tpu_sc as plsc

_N = 10000
_E = 640000
_H = 16
_NC, _NS = 2, 16
_NW = _NC * _NS
_EPW = _E // _NW
_CH = 2000
_NCHUNK = _EPW // _CH
_NPAD = 10240
_RPT = _NPAD // _NS

_MESH = plsc.VectorSubcoreMesh(core_axis_name="c", subcore_axis_name="s")


@functools.partial(
    pl.kernel,
    out_type=(
        jax.ShapeDtypeStruct((_NPAD,), jnp.float32),
        jax.ShapeDtypeStruct((_NPAD,), jnp.float32),
    ),
    mesh=_MESH,
    scratch_types=[
        pltpu.VMEM((_CH,), jnp.int32),
        pltpu.VMEM((_CH,), jnp.float32),
        pltpu.VMEM_SHARED((_NPAD,), jnp.float32),
    ],
    compiler_params=pltpu.CompilerParams(use_tc_tiling_on_sc=False),
)
def _sc_degree(dst_hbm, zeros_hbm, ones_hbm, d0_hbm, d1_hbm, idx_v, ones_v, acc_sh):
    c = lax.axis_index("c")
    s = lax.axis_index("s")
    wid = s * _NC + c
    pltpu.sync_copy(zeros_hbm.at[pl.ds(s * _RPT, _RPT)], acc_sh.at[pl.ds(s * _RPT, _RPT)])
    pltpu.sync_copy(ones_hbm, ones_v)
    plsc.subcore_barrier()
    base = wid * _EPW
    for k in range(_NCHUNK):
        pltpu.sync_copy(dst_hbm.at[pl.ds(base + k * _CH, _CH)], idx_v)
        pltpu.sync_copy(ones_v, acc_sh.at[idx_v], add=True)
    plsc.subcore_barrier()

    @pl.when(c == 0)
    def _():
        pltpu.sync_copy(acc_sh.at[pl.ds(s * _RPT, _RPT)], d0_hbm.at[pl.ds(s * _RPT, _RPT)])

    @pl.when(c == 1)
    def _():
        pltpu.sync_copy(acc_sh.at[pl.ds(s * _RPT, _RPT)], d1_hbm.at[pl.ds(s * _RPT, _RPT)])


@functools.partial(
    pl.kernel,
    out_type=(
        jax.ShapeDtypeStruct((_NPAD, _H), jnp.float32),
        jax.ShapeDtypeStruct((_NPAD, _H), jnp.float32),
    ),
    mesh=_MESH,
    scratch_types=[
        pltpu.VMEM((_CH,), jnp.int32),
        pltpu.VMEM((_CH,), jnp.int32),
        pltpu.VMEM((_CH, _H), jnp.float32),
        pltpu.VMEM_SHARED((_NPAD, _H), jnp.float32),
        pltpu.SemaphoreType.DMA,
    ],
    compiler_params=pltpu.CompilerParams(use_tc_tiling_on_sc=False),
)
def _sc_edge(t_hbm, src_hbm, dst_hbm, p0_hbm, p1_hbm,
             sidx_v, didx_v, rows_v, acc_sh, sem):
    c = lax.axis_index("c")
    s = lax.axis_index("s")
    wid = s * _NC + c
    pltpu.sync_copy(t_hbm.at[pl.ds(s * _RPT, _RPT)], acc_sh.at[pl.ds(s * _RPT, _RPT)])
    plsc.subcore_barrier()
    base = wid * _EPW
    for k in range(_NCHUNK):
        pltpu.sync_copy(src_hbm.at[pl.ds(base + k * _CH, _CH)], sidx_v)
        pltpu.sync_copy(dst_hbm.at[pl.ds(base + k * _CH, _CH)], didx_v)
        pltpu.async_copy(t_hbm.at[sidx_v], rows_v, sem).wait()
        pltpu.sync_copy(rows_v, acc_sh.at[didx_v], add=True)
    plsc.subcore_barrier()

    @pl.when(c == 0)
    def _():
        pltpu.sync_copy(acc_sh.at[pl.ds(s * _RPT, _RPT)], p0_hbm.at[pl.ds(s * _RPT, _RPT)])

    @pl.when(c == 1)
    def _():
        pltpu.sync_copy(acc_sh.at[pl.ds(s * _RPT, _RPT)], p1_hbm.at[pl.ds(s * _RPT, _RPT)])


_BLKP = 1024


_F_IN = 1433
_KMAIN = 1408
_KCH = 176
_KTAIL = _F_IN - _KMAIN
_NKCH = _KMAIN // _KCH


def _mm_scale_body(d0_ref, d1_ref, xt_hbm, xtail_ref, wm_ref, wt_ref, o_ref,
                   xb0, xb1, sem0, sem1):
    xbs = (xb0, xb1)
    sems = (sem0, sem1)
    desc = pltpu.async_copy(xt_hbm.at[pl.ds(0, _KCH)], xb0, sem0)
    for i in range(_NKCH):
        b = i & 1
        nxt = None
        if i + 1 < _NKCH:
            nxt = pltpu.async_copy(
                xt_hbm.at[pl.ds((i + 1) * _KCH, _KCH)], xbs[(i + 1) & 1],
                sems[(i + 1) & 1])
        desc.wait()
        part = lax.dot_general(
            xbs[b][...], wm_ref[pl.ds(i * _KCH, _KCH), :],
            (((0,), (0,)), ((), ())), preferred_element_type=jnp.float32)
        if i == 0:
            o_ref[pl.ds(0, _N), :] = part
        else:
            o_ref[pl.ds(0, _N), :] += part
        desc = nxt
    tail = lax.dot_general(
        xtail_ref[...], wt_ref[...], (((0,), (0,)), ((), ())),
        preferred_element_type=jnp.float32)
    dinv = lax.rsqrt(d0_ref[pl.ds(0, _N), :] + d1_ref[pl.ds(0, _N), :] + 1.0)
    o_ref[pl.ds(0, _N), :] = (o_ref[pl.ds(0, _N), :] + tail) * dinv
    o_ref[pl.ds(_N, _NPAD - _N), :] = jnp.zeros((_NPAD - _N, _H), jnp.float32)


def _tc_mm_scale(d0, d1, xt_main, xt_tail, W1m, W1t):
    return pl.pallas_call(
        _mm_scale_body,
        grid=(1,),
        in_specs=[
            pl.BlockSpec((_NPAD, 1), lambda i: (0, 0)),
            pl.BlockSpec((_NPAD, 1), lambda i: (0, 0)),
            pl.BlockSpec(memory_space=pltpu.MemorySpace.HBM),
            pl.BlockSpec((_KTAIL, _N), lambda i: (0, 0)),
            pl.BlockSpec((_KMAIN, _H), lambda i: (0, 0)),
            pl.BlockSpec((_KTAIL, _H), lambda i: (0, 0)),
        ],
        out_specs=pl.BlockSpec((_NPAD, _H), lambda i: (0, 0)),
        out_shape=jax.ShapeDtypeStruct((_NPAD, _H), jnp.float32),
        scratch_shapes=[
            pltpu.VMEM((_KCH, _N), jnp.float32),
            pltpu.VMEM((_KCH, _N), jnp.float32),
            pltpu.SemaphoreType.DMA,
            pltpu.SemaphoreType.DMA,
        ],
        compiler_params=pltpu.CompilerParams(fuse_transposed_lhs_in_matmul=True),
    )(d0, d1, xt_main, xt_tail, W1m, W1t)


def _layer2_body(d0_ref, d1_ref, p0_ref, p1_ref, t1_ref, b1_ref, w2_ref, o_ref):
    dinv = lax.rsqrt(d0_ref[...] + d1_ref[...] + 1.0)
    agg = p0_ref[...] + p1_ref[...] - t1_ref[...]
    h1 = jnp.maximum(dinv * agg + b1_ref[...], 0.0)
    h2 = jnp.dot(h1, w2_ref[...], preferred_element_type=jnp.float32)
    o_ref[...] = dinv * h2


def _tc_layer2(d0, d1, p0, p1, t1, b1r, W2p):
    return pl.pallas_call(
        _layer2_body,
        grid=(_NPAD // _BLKP,),
        in_specs=[
            pl.BlockSpec((_BLKP, 1), lambda i: (i, 0)),
            pl.BlockSpec((_BLKP, 1), lambda i: (i, 0)),
            pl.BlockSpec((_BLKP, _H), lambda i: (i, 0)),
            pl.BlockSpec((_BLKP, _H), lambda i: (i, 0)),
            pl.BlockSpec((_BLKP, _H), lambda i: (i, 0)),
            pl.BlockSpec((1, _H), lambda i: (0, 0)),
            pl.BlockSpec((_H, _H), lambda i: (0, 0)),
        ],
        out_specs=pl.BlockSpec((_BLKP, _H), lambda i: (i, 0)),
        out_shape=jax.ShapeDtypeStruct((_NPAD, _H), jnp.float32),
    )(d0, d1, p0, p1, t1, b1r, W2p)


def _final_body(d0_ref, d1_ref, q0_ref, q1_ref, t2_ref, b2_ref, o_ref):
    dinv = lax.rsqrt(d0_ref[...] + d1_ref[...] + 1.0)
    z = dinv * (q0_ref[...] + q1_ref[...] - t2_ref[...]) + b2_ref[...]
    col = lax.broadcasted_iota(jnp.int32, z.shape, 1)
    valid = col < 7
    zm = jnp.where(valid, z, -jnp.inf)
    m = jnp.max(zm, axis=1, keepdims=True)
    e = jnp.where(valid, jnp.exp(z - m), 0.0)
    ssum = jnp.sum(e, axis=1, keepdims=True)
    o_ref[...] = z - m - jnp.log(ssum)


def _tc_final(d0, d1, q0, q1, t2, b2p):
    return pl.pallas_call(
        _final_body,
        grid=(_NPAD // _BLKP,),
        in_specs=[
            pl.BlockSpec((_BLKP, 1), lambda i: (i, 0)),
            pl.BlockSpec((_BLKP, 1), lambda i: (i, 0)),
            pl.BlockSpec((_BLKP, _H), lambda i: (i, 0)),
            pl.BlockSpec((_BLKP, _H), lambda i: (i, 0)),
            pl.BlockSpec((_BLKP, _H), lambda i: (i, 0)),
            pl.BlockSpec((1, _H), lambda i: (0, 0)),
        ],
        out_specs=pl.BlockSpec((_BLKP, _H), lambda i: (i, 0)),
        out_shape=jax.ShapeDtypeStruct((_NPAD, _H), jnp.float32),
    )(d0, d1, q0, q1, t2, b2p)


def kernel(x, edge_index, W1, b1, W2, b2):
    src2 = edge_index[0]
    dst2 = edge_index[1]
    zeros_n = jnp.zeros((_NPAD,), jnp.float32)
    ones_ch = jnp.ones((_CH,), jnp.float32)

    d0p, d1p = _sc_degree(dst2, zeros_n, ones_ch)
    d0 = d0p.reshape(_NPAD, 1)
    d1 = d1p.reshape(_NPAD, 1)

    xt_main = x[:, :_KMAIN].T
    xt_tail = x[:, _KMAIN:].T
    t1 = _tc_mm_scale(d0, d1, xt_main, xt_tail, W1[:_KMAIN], W1[_KMAIN:])
    p0, p1 = _sc_edge(t1, src2, dst2)

    b1r = b1.reshape(1, _H)
    W2p = jnp.zeros((_H, _H), jnp.float32).at[:, : W2.shape[1]].set(W2)
    t2 = _tc_layer2(d0, d1, p0, p1, t1, b1r, W2p)

    q0, q1 = _sc_edge(t2, src2, dst2)

    b2p = jnp.zeros((1, _H), jnp.float32).at[0, : b2.shape[0]].set(b2)
    out = _tc_final(d0, d1, q0, q1, t2, b2p)
    return out[:_N, :7]


# speedup vs baseline: 1.3092x; 1.3092x over previous
import functools

import jax
import jax.numpy as jnp
from jax import lax
from jax.experimental import pallas as pl
from jax.experimental.pallas import tpu as pltpu
from jax.experimental.pallas import tpu_sc as plsc

_N = 10000
_E = 640000
_H = 16
_NC, _NS = 2, 16
_NW = _NC * _NS
_EPW = _E // _NW
_CH = 2000
_NCHUNK = _EPW // _CH
_NPAD = 10240
_RPT = _NPAD // _NS

_MESH = plsc.VectorSubcoreMesh(core_axis_name="c", subcore_axis_name="s")


@functools.partial(
    pl.kernel,
    out_type=(
        jax.ShapeDtypeStruct((_NPAD,), jnp.float32),
        jax.ShapeDtypeStruct((_NPAD,), jnp.float32),
    ),
    mesh=_MESH,
    scratch_types=[
        pltpu.VMEM((_CH,), jnp.int32),
        pltpu.VMEM((_CH,), jnp.float32),
        pltpu.VMEM_SHARED((_NPAD,), jnp.float32),
    ],
    compiler_params=pltpu.CompilerParams(use_tc_tiling_on_sc=False),
)
def _sc_degree(dst_hbm, zeros_hbm, ones_hbm, d0_hbm, d1_hbm, idx_v, ones_v, acc_sh):
    c = lax.axis_index("c")
    s = lax.axis_index("s")
    wid = s * _NC + c
    pltpu.sync_copy(zeros_hbm.at[pl.ds(s * _RPT, _RPT)], acc_sh.at[pl.ds(s * _RPT, _RPT)])
    pltpu.sync_copy(ones_hbm, ones_v)
    plsc.subcore_barrier()
    base = wid * _EPW
    for k in range(_NCHUNK):
        pltpu.sync_copy(dst_hbm.at[pl.ds(base + k * _CH, _CH)], idx_v)
        pltpu.sync_copy(ones_v, acc_sh.at[idx_v], add=True)
    plsc.subcore_barrier()

    @pl.when(c == 0)
    def _():
        pltpu.sync_copy(acc_sh.at[pl.ds(s * _RPT, _RPT)], d0_hbm.at[pl.ds(s * _RPT, _RPT)])

    @pl.when(c == 1)
    def _():
        pltpu.sync_copy(acc_sh.at[pl.ds(s * _RPT, _RPT)], d1_hbm.at[pl.ds(s * _RPT, _RPT)])


@functools.partial(
    pl.kernel,
    out_type=(
        jax.ShapeDtypeStruct((_NPAD, _H), jnp.float32),
        jax.ShapeDtypeStruct((_NPAD, _H), jnp.float32),
    ),
    mesh=_MESH,
    scratch_types=[
        pltpu.VMEM((_CH,), jnp.int32),
        pltpu.VMEM((_CH,), jnp.int32),
        pltpu.VMEM((_CH,), jnp.int32),
        pltpu.VMEM((_CH,), jnp.int32),
        pltpu.VMEM((_CH, _H), jnp.float32),
        pltpu.VMEM((_CH, _H), jnp.float32),
        pltpu.VMEM_SHARED((_NPAD, _H), jnp.float32),
        pltpu.SemaphoreType.DMA,
        pltpu.SemaphoreType.DMA,
    ],
    compiler_params=pltpu.CompilerParams(use_tc_tiling_on_sc=False),
)
def _sc_edge(t_hbm, src_hbm, dst_hbm, p0_hbm, p1_hbm,
             sidx0, sidx1, didx0, didx1, rows0, rows1, acc_sh, sem0, sem1):
    c = lax.axis_index("c")
    s = lax.axis_index("s")
    wid = s * _NC + c
    pltpu.sync_copy(t_hbm.at[pl.ds(s * _RPT, _RPT)], acc_sh.at[pl.ds(s * _RPT, _RPT)])
    plsc.subcore_barrier()
    base = wid * _EPW
    sidx = (sidx0, sidx1)
    didx = (didx0, didx1)
    rows = (rows0, rows1)
    sems = (sem0, sem1)
    pltpu.sync_copy(src_hbm.at[pl.ds(base, _CH)], sidx0)
    pltpu.sync_copy(dst_hbm.at[pl.ds(base, _CH)], didx0)
    desc = pltpu.async_copy(t_hbm.at[sidx0], rows0, sem0)
    for k in range(_NCHUNK):
        b = k & 1
        nb = (k + 1) & 1
        nxt = None
        if k + 1 < _NCHUNK:
            off = base + (k + 1) * _CH
            pltpu.sync_copy(src_hbm.at[pl.ds(off, _CH)], sidx[nb])
            pltpu.sync_copy(dst_hbm.at[pl.ds(off, _CH)], didx[nb])
            nxt = pltpu.async_copy(t_hbm.at[sidx[nb]], rows[nb], sems[nb])
        desc.wait()
        pltpu.sync_copy(rows[b], acc_sh.at[didx[b]], add=True)
        desc = nxt
    plsc.subcore_barrier()

    @pl.when(c == 0)
    def _():
        pltpu.sync_copy(acc_sh.at[pl.ds(s * _RPT, _RPT)], p0_hbm.at[pl.ds(s * _RPT, _RPT)])

    @pl.when(c == 1)
    def _():
        pltpu.sync_copy(acc_sh.at[pl.ds(s * _RPT, _RPT)], p1_hbm.at[pl.ds(s * _RPT, _RPT)])


_BLKP = 1024


_F_IN = 1433
_KMAIN = 1408
_KCH = 176
_KTAIL = _F_IN - _KMAIN
_NKCH = _KMAIN // _KCH


_NBUF = 3


def _mm_body(xt_hbm, xtail_ref, wm_ref, wt_ref, o_ref,
             xb0, xb1, xb2, sem0, sem1, sem2):
    xbs = (xb0, xb1, xb2)
    sems = (sem0, sem1, sem2)
    descs = [None] * _NKCH
    for i in range(min(_NBUF, _NKCH)):
        descs[i] = pltpu.async_copy(
            xt_hbm.at[pl.ds(i * _KCH, _KCH)], xbs[i % _NBUF], sems[i % _NBUF])
    for i in range(_NKCH):
        b = i % _NBUF
        descs[i].wait()
        part = lax.dot_general(
            xbs[b][...], wm_ref[pl.ds(i * _KCH, _KCH), :],
            (((0,), (0,)), ((), ())), preferred_element_type=jnp.float32)
        if i == 0:
            o_ref[pl.ds(0, _N), :] = part
        else:
            o_ref[pl.ds(0, _N), :] += part
        nx = i + _NBUF
        if nx < _NKCH:
            descs[nx] = pltpu.async_copy(
                xt_hbm.at[pl.ds(nx * _KCH, _KCH)], xbs[b], sems[b])
    tail = lax.dot_general(
        xtail_ref[...], wt_ref[...], (((0,), (0,)), ((), ())),
        preferred_element_type=jnp.float32)
    o_ref[pl.ds(0, _N), :] += tail
    o_ref[pl.ds(_N, _NPAD - _N), :] = jnp.zeros((_NPAD - _N, _H), jnp.float32)


def _tc_mm(xt, xt_tail, W1m, W1t):
    return pl.pallas_call(
        _mm_body,
        grid=(1,),
        in_specs=[
            pl.BlockSpec(memory_space=pltpu.MemorySpace.HBM),
            pl.BlockSpec((_KTAIL, _N), lambda i: (0, 0)),
            pl.BlockSpec((_KMAIN, _H), lambda i: (0, 0)),
            pl.BlockSpec((_KTAIL, _H), lambda i: (0, 0)),
        ],
        out_specs=pl.BlockSpec((_NPAD, _H), lambda i: (0, 0)),
        out_shape=jax.ShapeDtypeStruct((_NPAD, _H), jnp.float32),
        scratch_shapes=[
            pltpu.VMEM((_KCH, _N), jnp.float32),
            pltpu.VMEM((_KCH, _N), jnp.float32),
            pltpu.VMEM((_KCH, _N), jnp.float32),
            pltpu.SemaphoreType.DMA,
            pltpu.SemaphoreType.DMA,
            pltpu.SemaphoreType.DMA,
        ],
        compiler_params=pltpu.CompilerParams(fuse_transposed_lhs_in_matmul=True),
    )(xt, xt_tail, W1m, W1t)


def _scale_body(d0_ref, d1_ref, h_ref, o_ref):
    dinv = lax.rsqrt(d0_ref[...] + d1_ref[...] + 1.0)
    o_ref[...] = h_ref[...] * dinv


def _tc_scale(d0, d1, h):
    return pl.pallas_call(
        _scale_body,
        grid=(_NPAD // _BLKP,),
        in_specs=[
            pl.BlockSpec((_BLKP, 1), lambda i: (i, 0)),
            pl.BlockSpec((_BLKP, 1), lambda i: (i, 0)),
            pl.BlockSpec((_BLKP, _H), lambda i: (i, 0)),
        ],
        out_specs=pl.BlockSpec((_BLKP, _H), lambda i: (i, 0)),
        out_shape=jax.ShapeDtypeStruct((_NPAD, _H), jnp.float32),
    )(d0, d1, h)


def _layer2_body(d0_ref, d1_ref, p0_ref, p1_ref, t1_ref, b1_ref, w2_ref, o_ref):
    dinv = lax.rsqrt(d0_ref[...] + d1_ref[...] + 1.0)
    agg = p0_ref[...] + p1_ref[...] - t1_ref[...]
    h1 = jnp.maximum(dinv * agg + b1_ref[...], 0.0)
    h2 = jnp.dot(h1, w2_ref[...], preferred_element_type=jnp.float32)
    o_ref[...] = dinv * h2


def _tc_layer2(d0, d1, p0, p1, t1, b1r, W2p):
    return pl.pallas_call(
        _layer2_body,
        grid=(_NPAD // _BLKP,),
        in_specs=[
            pl.BlockSpec((_BLKP, 1), lambda i: (i, 0)),
            pl.BlockSpec((_BLKP, 1), lambda i: (i, 0)),
            pl.BlockSpec((_BLKP, _H), lambda i: (i, 0)),
            pl.BlockSpec((_BLKP, _H), lambda i: (i, 0)),
            pl.BlockSpec((_BLKP, _H), lambda i: (i, 0)),
            pl.BlockSpec((1, _H), lambda i: (0, 0)),
            pl.BlockSpec((_H, _H), lambda i: (0, 0)),
        ],
        out_specs=pl.BlockSpec((_BLKP, _H), lambda i: (i, 0)),
        out_shape=jax.ShapeDtypeStruct((_NPAD, _H), jnp.float32),
    )(d0, d1, p0, p1, t1, b1r, W2p)


def _final_body(d0_ref, d1_ref, q0_ref, q1_ref, t2_ref, b2_ref, o_ref):
    dinv = lax.rsqrt(d0_ref[...] + d1_ref[...] + 1.0)
    z = dinv * (q0_ref[...] + q1_ref[...] - t2_ref[...]) + b2_ref[...]
    col = lax.broadcasted_iota(jnp.int32, z.shape, 1)
    valid = col < 7
    zm = jnp.where(valid, z, -jnp.inf)
    m = jnp.max(zm, axis=1, keepdims=True)
    e = jnp.where(valid, jnp.exp(z - m), 0.0)
    ssum = jnp.sum(e, axis=1, keepdims=True)
    o_ref[...] = z - m - jnp.log(ssum)


def _tc_final(d0, d1, q0, q1, t2, b2p):
    return pl.pallas_call(
        _final_body,
        grid=(_NPAD // _BLKP,),
        in_specs=[
            pl.BlockSpec((_BLKP, 1), lambda i: (i, 0)),
            pl.BlockSpec((_BLKP, 1), lambda i: (i, 0)),
            pl.BlockSpec((_BLKP, _H), lambda i: (i, 0)),
            pl.BlockSpec((_BLKP, _H), lambda i: (i, 0)),
            pl.BlockSpec((_BLKP, _H), lambda i: (i, 0)),
            pl.BlockSpec((1, _H), lambda i: (0, 0)),
        ],
        out_specs=pl.BlockSpec((_BLKP, _H), lambda i: (i, 0)),
        out_shape=jax.ShapeDtypeStruct((_NPAD, _H), jnp.float32),
    )(d0, d1, q0, q1, t2, b2p)


def kernel(x, edge_index, W1, b1, W2, b2):
    src2 = edge_index[0]
    dst2 = edge_index[1]
    zeros_n = jnp.zeros((_NPAD,), jnp.float32)
    ones_ch = jnp.ones((_CH,), jnp.float32)

    d0p, d1p = _sc_degree(dst2, zeros_n, ones_ch)
    d0 = d0p.reshape(_NPAD, 1)
    d1 = d1p.reshape(_NPAD, 1)

    xt = x.T
    xt_tail = x[:, _KMAIN:].T
    h1 = _tc_mm(xt, xt_tail, W1[:_KMAIN], W1[_KMAIN:])
    t1 = _tc_scale(d0, d1, h1)
    p0, p1 = _sc_edge(t1, src2, dst2)

    b1r = b1.reshape(1, _H)
    W2p = jnp.zeros((_H, _H), jnp.float32).at[:, : W2.shape[1]].set(W2)
    t2 = _tc_layer2(d0, d1, p0, p1, t1, b1r, W2p)

    q0, q1 = _sc_edge(t2, src2, dst2)

    b2p = jnp.zeros((1, _H), jnp.float32).at[0, : b2.shape[0]].set(b2)
    out = _tc_final(d0, d1, q0, q1, t2, b2p)
    return out[:_N, :7]


# speedup vs baseline: 1.3994x; 1.0689x over previous
import functools

import jax
import jax.numpy as jnp
from jax import lax
from jax.experimental import pallas as pl
from jax.experimental.pallas import tpu as pltpu
from jax.experimental.pallas import tpu_sc as plsc

_N = 10000
_E = 640000
_H = 16
_NC, _NS = 2, 16
_NW = _NC * _NS
_EPW = _E // _NW
_CH = 2000
_NCHUNK = _EPW // _CH
_NPAD = 10240
_RPT = _NPAD // _NS

_MESH = plsc.VectorSubcoreMesh(core_axis_name="c", subcore_axis_name="s")


@functools.partial(
    pl.kernel,
    out_type=(
        jax.ShapeDtypeStruct((_NPAD,), jnp.float32),
        jax.ShapeDtypeStruct((_NPAD,), jnp.float32),
    ),
    mesh=_MESH,
    scratch_types=[
        pltpu.VMEM((_CH,), jnp.int32),
        pltpu.VMEM((_CH,), jnp.float32),
        pltpu.VMEM_SHARED((_NPAD,), jnp.float32),
    ],
    compiler_params=pltpu.CompilerParams(use_tc_tiling_on_sc=False),
)
def _sc_degree(dst_hbm, zeros_hbm, ones_hbm, d0_hbm, d1_hbm, idx_v, ones_v, acc_sh):
    c = lax.axis_index("c")
    s = lax.axis_index("s")
    wid = s * _NC + c
    pltpu.sync_copy(zeros_hbm.at[pl.ds(s * _RPT, _RPT)], acc_sh.at[pl.ds(s * _RPT, _RPT)])
    pltpu.sync_copy(ones_hbm, ones_v)
    plsc.subcore_barrier()
    base = wid * _EPW
    for k in range(_NCHUNK):
        pltpu.sync_copy(dst_hbm.at[pl.ds(base + k * _CH, _CH)], idx_v)
        pltpu.sync_copy(ones_v, acc_sh.at[idx_v], add=True)
    plsc.subcore_barrier()

    @pl.when(c == 0)
    def _():
        pltpu.sync_copy(acc_sh.at[pl.ds(s * _RPT, _RPT)], d0_hbm.at[pl.ds(s * _RPT, _RPT)])

    @pl.when(c == 1)
    def _():
        pltpu.sync_copy(acc_sh.at[pl.ds(s * _RPT, _RPT)], d1_hbm.at[pl.ds(s * _RPT, _RPT)])


@functools.partial(
    pl.kernel,
    out_type=(
        jax.ShapeDtypeStruct((_NPAD, _H), jnp.float32),
        jax.ShapeDtypeStruct((_NPAD, _H), jnp.float32),
    ),
    mesh=_MESH,
    scratch_types=[
        pltpu.VMEM((_CH,), jnp.int32),
        pltpu.VMEM((_CH,), jnp.int32),
        pltpu.VMEM((_CH,), jnp.int32),
        pltpu.VMEM((_CH,), jnp.int32),
        pltpu.VMEM((_CH, _H), jnp.float32),
        pltpu.VMEM((_CH, _H), jnp.float32),
        pltpu.VMEM_SHARED((_NPAD, _H), jnp.float32),
        pltpu.SemaphoreType.DMA,
        pltpu.SemaphoreType.DMA,
    ],
    compiler_params=pltpu.CompilerParams(use_tc_tiling_on_sc=False),
)
def _sc_edge(t_hbm, src_hbm, dst_hbm, p0_hbm, p1_hbm,
             sidx0, sidx1, didx0, didx1, rows0, rows1, acc_sh, sem0, sem1):
    c = lax.axis_index("c")
    s = lax.axis_index("s")
    wid = s * _NC + c
    pltpu.sync_copy(t_hbm.at[pl.ds(s * _RPT, _RPT)], acc_sh.at[pl.ds(s * _RPT, _RPT)])
    plsc.subcore_barrier()
    base = wid * _EPW
    sidx = (sidx0, sidx1)
    didx = (didx0, didx1)
    rows = (rows0, rows1)
    sems = (sem0, sem1)
    pltpu.sync_copy(src_hbm.at[pl.ds(base, _CH)], sidx0)
    pltpu.sync_copy(dst_hbm.at[pl.ds(base, _CH)], didx0)
    desc = pltpu.async_copy(t_hbm.at[sidx0], rows0, sem0)
    for k in range(_NCHUNK):
        b = k & 1
        nb = (k + 1) & 1
        nxt = None
        if k + 1 < _NCHUNK:
            off = base + (k + 1) * _CH
            pltpu.sync_copy(src_hbm.at[pl.ds(off, _CH)], sidx[nb])
            pltpu.sync_copy(dst_hbm.at[pl.ds(off, _CH)], didx[nb])
            nxt = pltpu.async_copy(t_hbm.at[sidx[nb]], rows[nb], sems[nb])
        desc.wait()
        pltpu.sync_copy(rows[b], acc_sh.at[didx[b]], add=True)
        desc = nxt
    plsc.subcore_barrier()

    @pl.when(c == 0)
    def _():
        pltpu.sync_copy(acc_sh.at[pl.ds(s * _RPT, _RPT)], p0_hbm.at[pl.ds(s * _RPT, _RPT)])

    @pl.when(c == 1)
    def _():
        pltpu.sync_copy(acc_sh.at[pl.ds(s * _RPT, _RPT)], p1_hbm.at[pl.ds(s * _RPT, _RPT)])


_BLKP = 1024


_F_IN = 1433
_KMAIN = 1408
_KCH = 176
_KTAIL = _F_IN - _KMAIN
_NKCH = _KMAIN // _KCH


def _mm_body(xt_ref, xtail_ref, wm_ref, wt_ref, o_ref):
    i = pl.program_id(0)
    part = lax.dot_general(
        xt_ref[...], wm_ref[...], (((0,), (0,)), ((), ())),
        preferred_element_type=jnp.float32)

    @pl.when(i == 0)
    def _():
        o_ref[pl.ds(0, _N), :] = part

    @pl.when(i > 0)
    def _():
        o_ref[pl.ds(0, _N), :] += part

    @pl.when(i == _NKCH - 1)
    def _():
        tail = lax.dot_general(
            xtail_ref[...], wt_ref[...], (((0,), (0,)), ((), ())),
            preferred_element_type=jnp.float32)
        o_ref[pl.ds(0, _N), :] += tail
        o_ref[pl.ds(_N, _NPAD - _N), :] = jnp.zeros((_NPAD - _N, _H), jnp.float32)


def _tc_mm(xt, xt_tail, W1m, W1t):
    return pl.pallas_call(
        _mm_body,
        grid=(_NKCH,),
        in_specs=[
            pl.BlockSpec((_KCH, _N), lambda i: (i, 0)),
            pl.BlockSpec((_KTAIL, _N), lambda i: (0, 0)),
            pl.BlockSpec((_KCH, _H), lambda i: (i, 0)),
            pl.BlockSpec((_KTAIL, _H), lambda i: (0, 0)),
        ],
        out_specs=pl.BlockSpec((_NPAD, _H), lambda i: (0, 0)),
        out_shape=jax.ShapeDtypeStruct((_NPAD, _H), jnp.float32),
        compiler_params=pltpu.CompilerParams(fuse_transposed_lhs_in_matmul=True),
    )(xt, xt_tail, W1m, W1t)


def _scale_body(d0_ref, d1_ref, h_ref, o_ref):
    dinv = lax.rsqrt(d0_ref[...] + d1_ref[...] + 1.0)
    o_ref[...] = h_ref[...] * dinv


def _tc_scale(d0, d1, h):
    return pl.pallas_call(
        _scale_body,
        grid=(_NPAD // _BLKP,),
        in_specs=[
            pl.BlockSpec((_BLKP, 1), lambda i: (i, 0)),
            pl.BlockSpec((_BLKP, 1), lambda i: (i, 0)),
            pl.BlockSpec((_BLKP, _H), lambda i: (i, 0)),
        ],
        out_specs=pl.BlockSpec((_BLKP, _H), lambda i: (i, 0)),
        out_shape=jax.ShapeDtypeStruct((_NPAD, _H), jnp.float32),
    )(d0, d1, h)


def _layer2_body(d0_ref, d1_ref, p0_ref, p1_ref, t1_ref, b1_ref, w2_ref, o_ref):
    dinv = lax.rsqrt(d0_ref[...] + d1_ref[...] + 1.0)
    agg = p0_ref[...] + p1_ref[...] - t1_ref[...]
    h1 = jnp.maximum(dinv * agg + b1_ref[...], 0.0)
    h2 = jnp.dot(h1, w2_ref[...], preferred_element_type=jnp.float32)
    o_ref[...] = dinv * h2


def _tc_layer2(d0, d1, p0, p1, t1, b1r, W2p):
    return pl.pallas_call(
        _layer2_body,
        grid=(_NPAD // _BLKP,),
        in_specs=[
            pl.BlockSpec((_BLKP, 1), lambda i: (i, 0)),
            pl.BlockSpec((_BLKP, 1), lambda i: (i, 0)),
            pl.BlockSpec((_BLKP, _H), lambda i: (i, 0)),
            pl.BlockSpec((_BLKP, _H), lambda i: (i, 0)),
            pl.BlockSpec((_BLKP, _H), lambda i: (i, 0)),
            pl.BlockSpec((1, _H), lambda i: (0, 0)),
            pl.BlockSpec((_H, _H), lambda i: (0, 0)),
        ],
        out_specs=pl.BlockSpec((_BLKP, _H), lambda i: (i, 0)),
        out_shape=jax.ShapeDtypeStruct((_NPAD, _H), jnp.float32),
    )(d0, d1, p0, p1, t1, b1r, W2p)


def _final_body(d0_ref, d1_ref, q0_ref, q1_ref, t2_ref, b2_ref, o_ref):
    dinv = lax.rsqrt(d0_ref[...] + d1_ref[...] + 1.0)
    z = dinv * (q0_ref[...] + q1_ref[...] - t2_ref[...]) + b2_ref[...]
    col = lax.broadcasted_iota(jnp.int32, z.shape, 1)
    valid = col < 7
    zm = jnp.where(valid, z, -jnp.inf)
    m = jnp.max(zm, axis=1, keepdims=True)
    e = jnp.where(valid, jnp.exp(z - m), 0.0)
    ssum = jnp.sum(e, axis=1, keepdims=True)
    o_ref[...] = z - m - jnp.log(ssum)


def _tc_final(d0, d1, q0, q1, t2, b2p):
    return pl.pallas_call(
        _final_body,
        grid=(_NPAD // _BLKP,),
        in_specs=[
            pl.BlockSpec((_BLKP, 1), lambda i: (i, 0)),
            pl.BlockSpec((_BLKP, 1), lambda i: (i, 0)),
            pl.BlockSpec((_BLKP, _H), lambda i: (i, 0)),
            pl.BlockSpec((_BLKP, _H), lambda i: (i, 0)),
            pl.BlockSpec((_BLKP, _H), lambda i: (i, 0)),
            pl.BlockSpec((1, _H), lambda i: (0, 0)),
        ],
        out_specs=pl.BlockSpec((_BLKP, _H), lambda i: (i, 0)),
        out_shape=jax.ShapeDtypeStruct((_NPAD, _H), jnp.float32),
    )(d0, d1, q0, q1, t2, b2p)


def kernel(x, edge_index, W1, b1, W2, b2):
    src2 = edge_index[0]
    dst2 = edge_index[1]
    zeros_n = jnp.zeros((_NPAD,), jnp.float32)
    ones_ch = jnp.ones((_CH,), jnp.float32)

    d0p, d1p = _sc_degree(dst2, zeros_n, ones_ch)
    d0 = d0p.reshape(_NPAD, 1)
    d1 = d1p.reshape(_NPAD, 1)

    xt = x.T
    xt_tail = x[:, _KMAIN:].T
    h1 = _tc_mm(xt, xt_tail, W1[:_KMAIN], W1[_KMAIN:])
    t1 = _tc_scale(d0, d1, h1)
    p0, p1 = _sc_edge(t1, src2, dst2)

    b1r = b1.reshape(1, _H)
    W2p = jnp.zeros((_H, _H), jnp.float32).at[:, : W2.shape[1]].set(W2)
    t2 = _tc_layer2(d0, d1, p0, p1, t1, b1r, W2p)

    q0, q1 = _sc_edge(t2, src2, dst2)

    b2p = jnp.zeros((1, _H), jnp.float32).at[0, : b2.shape[0]].set(b2)
    out = _tc_final(d0, d1, q0, q1, t2, b2p)
    return out[:_N, :7]


# speedup vs baseline: 1.6534x; 1.1815x over previous
import functools

import jax
import jax.numpy as jnp
from jax import lax
from jax.experimental import pallas as pl
from jax.experimental.pallas import tpu as pltpu
from jax.experimental.pallas import tpu_sc as plsc

_N = 10000
_E = 640000
_H = 16
_NC, _NS = 2, 16
_NW = _NC * _NS
_EPW = _E // _NW
_CH = 2000
_NCHUNK = _EPW // _CH
_NPAD = 10240
_RPT = _NPAD // _NS

_MESH = plsc.VectorSubcoreMesh(core_axis_name="c", subcore_axis_name="s")


@functools.partial(
    pl.kernel,
    out_type=(
        jax.ShapeDtypeStruct((_NPAD,), jnp.float32),
        jax.ShapeDtypeStruct((_NPAD,), jnp.float32),
    ),
    mesh=_MESH,
    scratch_types=[
        pltpu.VMEM((_CH,), jnp.int32),
        pltpu.VMEM((_CH,), jnp.float32),
        pltpu.VMEM_SHARED((_NPAD,), jnp.float32),
    ],
    compiler_params=pltpu.CompilerParams(use_tc_tiling_on_sc=False),
)
def _sc_degree(dst_hbm, zeros_hbm, ones_hbm, d0_hbm, d1_hbm, idx_v, ones_v, acc_sh):
    c = lax.axis_index("c")
    s = lax.axis_index("s")
    wid = s * _NC + c
    pltpu.sync_copy(zeros_hbm.at[pl.ds(s * _RPT, _RPT)], acc_sh.at[pl.ds(s * _RPT, _RPT)])
    pltpu.sync_copy(ones_hbm, ones_v)
    plsc.subcore_barrier()
    base = wid * _EPW
    for k in range(_NCHUNK):
        pltpu.sync_copy(dst_hbm.at[pl.ds(base + k * _CH, _CH)], idx_v)
        pltpu.sync_copy(ones_v, acc_sh.at[idx_v], add=True)
    plsc.subcore_barrier()

    @pl.when(c == 0)
    def _():
        pltpu.sync_copy(acc_sh.at[pl.ds(s * _RPT, _RPT)], d0_hbm.at[pl.ds(s * _RPT, _RPT)])

    @pl.when(c == 1)
    def _():
        pltpu.sync_copy(acc_sh.at[pl.ds(s * _RPT, _RPT)], d1_hbm.at[pl.ds(s * _RPT, _RPT)])


@functools.partial(
    pl.kernel,
    out_type=(
        jax.ShapeDtypeStruct((_NPAD, _H), jnp.float32),
        jax.ShapeDtypeStruct((_NPAD, _H), jnp.float32),
    ),
    mesh=_MESH,
    scratch_types=[
        pltpu.VMEM((_CH,), jnp.int32),
        pltpu.VMEM((_CH,), jnp.int32),
        pltpu.VMEM((_CH,), jnp.int32),
        pltpu.VMEM((_CH,), jnp.int32),
        pltpu.VMEM((_CH, _H), jnp.float32),
        pltpu.VMEM((_CH, _H), jnp.float32),
        pltpu.VMEM_SHARED((_NPAD, _H), jnp.float32),
        pltpu.SemaphoreType.DMA,
        pltpu.SemaphoreType.DMA,
    ],
    compiler_params=pltpu.CompilerParams(use_tc_tiling_on_sc=False),
)
def _sc_edge(t_hbm, src_hbm, dst_hbm, p0_hbm, p1_hbm,
             sidx0, sidx1, didx0, didx1, rows0, rows1, acc_sh, sem0, sem1):
    c = lax.axis_index("c")
    s = lax.axis_index("s")
    wid = s * _NC + c
    pltpu.sync_copy(t_hbm.at[pl.ds(s * _RPT, _RPT)], acc_sh.at[pl.ds(s * _RPT, _RPT)])
    plsc.subcore_barrier()
    base = wid * _EPW
    sidx = (sidx0, sidx1)
    didx = (didx0, didx1)
    rows = (rows0, rows1)
    sems = (sem0, sem1)
    pltpu.sync_copy(src_hbm.at[pl.ds(base, _CH)], sidx0)
    pltpu.sync_copy(dst_hbm.at[pl.ds(base, _CH)], didx0)
    desc = pltpu.async_copy(t_hbm.at[sidx0], rows0, sem0)
    for k in range(_NCHUNK):
        b = k & 1
        nb = (k + 1) & 1
        nxt = None
        if k + 1 < _NCHUNK:
            off = base + (k + 1) * _CH
            pltpu.sync_copy(src_hbm.at[pl.ds(off, _CH)], sidx[nb])
            pltpu.sync_copy(dst_hbm.at[pl.ds(off, _CH)], didx[nb])
            nxt = pltpu.async_copy(t_hbm.at[sidx[nb]], rows[nb], sems[nb])
        desc.wait()
        pltpu.sync_copy(rows[b], acc_sh.at[didx[b]], add=True)
        desc = nxt
    plsc.subcore_barrier()

    @pl.when(c == 0)
    def _():
        pltpu.sync_copy(acc_sh.at[pl.ds(s * _RPT, _RPT)], p0_hbm.at[pl.ds(s * _RPT, _RPT)])

    @pl.when(c == 1)
    def _():
        pltpu.sync_copy(acc_sh.at[pl.ds(s * _RPT, _RPT)], p1_hbm.at[pl.ds(s * _RPT, _RPT)])


_BLKP = 1024


_F_IN = 1433
_KMAIN = 1408
_KCH = 176
_KTAIL = _F_IN - _KMAIN
_NKCH = _KMAIN // _KCH


def _mm_body(xt_ref, xtail_ref, wm_ref, wt_ref, o_ref):
    i = pl.program_id(0)
    part = lax.dot_general(
        xt_ref[...], wm_ref[...], (((0,), (0,)), ((), ())),
        preferred_element_type=jnp.float32)

    @pl.when(i == 0)
    def _():
        o_ref[pl.ds(0, _N), :] = part

    @pl.when(i > 0)
    def _():
        o_ref[pl.ds(0, _N), :] += part

    @pl.when(i == _NKCH - 1)
    def _():
        tail = lax.dot_general(
            xtail_ref[...], wt_ref[...], (((0,), (0,)), ((), ())),
            preferred_element_type=jnp.float32)
        o_ref[pl.ds(0, _N), :] += tail
        o_ref[pl.ds(_N, _NPAD - _N), :] = jnp.zeros((_NPAD - _N, _H), jnp.float32)


def _tc_mm(xt, xt_tail, W1m, W1t):
    return pl.pallas_call(
        _mm_body,
        grid=(_NKCH,),
        in_specs=[
            pl.BlockSpec((_KCH, _N), lambda i: (i, 0)),
            pl.BlockSpec((_KTAIL, _N), lambda i: (0, 0)),
            pl.BlockSpec((_KCH, _H), lambda i: (i, 0)),
            pl.BlockSpec((_KTAIL, _H), lambda i: (0, 0)),
        ],
        out_specs=pl.BlockSpec((_NPAD, _H), lambda i: (0, 0)),
        out_shape=jax.ShapeDtypeStruct((_NPAD, _H), jnp.float32),
        compiler_params=pltpu.CompilerParams(fuse_transposed_lhs_in_matmul=True),
    )(xt, xt_tail, W1m, W1t)


_NV = _NPAD // _H
_W = _H * _H


def _scale_body(d0_ref, d1_ref, p_ref, h_ref, o_ref):
    dinv16 = lax.rsqrt(d0_ref[...] + d1_ref[...] + 1.0)
    dinvb = jnp.dot(dinv16, p_ref[...], preferred_element_type=jnp.float32)
    o_ref[...] = h_ref[...] * dinvb


def _tc_scale(d0v, d1v, P, hv):
    return pl.pallas_call(
        _scale_body,
        grid=(1,),
        in_specs=[
            pl.BlockSpec((_NV, _H), lambda i: (0, 0)),
            pl.BlockSpec((_NV, _H), lambda i: (0, 0)),
            pl.BlockSpec((_H, _W), lambda i: (0, 0)),
            pl.BlockSpec((_NV, _W), lambda i: (0, 0)),
        ],
        out_specs=pl.BlockSpec((_NV, _W), lambda i: (0, 0)),
        out_shape=jax.ShapeDtypeStruct((_NV, _W), jnp.float32),
    )(d0v, d1v, P, hv)


def _layer2_body(d0_ref, d1_ref, p_ref, p0_ref, p1_ref, t1_ref, b1_ref, w2_ref, o_ref):
    dinv16 = lax.rsqrt(d0_ref[...] + d1_ref[...] + 1.0)
    dinvb = jnp.dot(dinv16, p_ref[...], preferred_element_type=jnp.float32)
    agg = p0_ref[...] + p1_ref[...] - t1_ref[...]
    h1 = jnp.maximum(dinvb * agg + b1_ref[...], 0.0)
    h2 = jnp.dot(h1, w2_ref[...], preferred_element_type=jnp.float32)
    o_ref[...] = dinvb * h2


def _tc_layer2(d0v, d1v, P, p0v, p1v, t1v, b1t, W2bd):
    return pl.pallas_call(
        _layer2_body,
        grid=(1,),
        in_specs=[
            pl.BlockSpec((_NV, _H), lambda i: (0, 0)),
            pl.BlockSpec((_NV, _H), lambda i: (0, 0)),
            pl.BlockSpec((_H, _W), lambda i: (0, 0)),
            pl.BlockSpec((_NV, _W), lambda i: (0, 0)),
            pl.BlockSpec((_NV, _W), lambda i: (0, 0)),
            pl.BlockSpec((_NV, _W), lambda i: (0, 0)),
            pl.BlockSpec((1, _W), lambda i: (0, 0)),
            pl.BlockSpec((_W, _W), lambda i: (0, 0)),
        ],
        out_specs=pl.BlockSpec((_NV, _W), lambda i: (0, 0)),
        out_shape=jax.ShapeDtypeStruct((_NV, _W), jnp.float32),
    )(d0v, d1v, P, p0v, p1v, t1v, b1t, W2bd)


def _final_body(d0_ref, d1_ref, p_ref, q0_ref, q1_ref, t2_ref, b2_ref, bc_ref, o_ref):
    dinv16 = lax.rsqrt(d0_ref[...] + d1_ref[...] + 1.0)
    dinvb = jnp.dot(dinv16, p_ref[...], preferred_element_type=jnp.float32)
    z = dinvb * (q0_ref[...] + q1_ref[...] - t2_ref[...]) + b2_ref[...]
    col = lax.broadcasted_iota(jnp.int32, z.shape, 1)
    cls = col & (_H - 1)
    valid = cls < 7
    gs = cls == 0
    zm = jnp.where(valid, z, -1e30)
    m = jnp.maximum(zm, pltpu.roll(zm, _W - 1, 1))
    m = jnp.maximum(m, pltpu.roll(m, _W - 2, 1))
    m = jnp.maximum(m, pltpu.roll(m, _W - 4, 1))
    mB = jnp.dot(jnp.where(gs, m, 0.0), bc_ref[...], preferred_element_type=jnp.float32)
    e = jnp.where(valid, jnp.exp(z - mB), 0.0)
    s = e + pltpu.roll(e, _W - 1, 1)
    s = s + pltpu.roll(s, _W - 2, 1)
    s = s + pltpu.roll(s, _W - 4, 1)
    sB = jnp.dot(jnp.where(gs, s, 0.0), bc_ref[...], preferred_element_type=jnp.float32)
    o_ref[...] = z - mB - jnp.log(sB)


def _tc_final(d0v, d1v, P, q0v, q1v, t2v, b2t, Bc):
    return pl.pallas_call(
        _final_body,
        grid=(1,),
        in_specs=[
            pl.BlockSpec((_NV, _H), lambda i: (0, 0)),
            pl.BlockSpec((_NV, _H), lambda i: (0, 0)),
            pl.BlockSpec((_H, _W), lambda i: (0, 0)),
            pl.BlockSpec((_NV, _W), lambda i: (0, 0)),
            pl.BlockSpec((_NV, _W), lambda i: (0, 0)),
            pl.BlockSpec((_NV, _W), lambda i: (0, 0)),
            pl.BlockSpec((1, _W), lambda i: (0, 0)),
            pl.BlockSpec((_W, _W), lambda i: (0, 0)),
        ],
        out_specs=pl.BlockSpec((_NV, _W), lambda i: (0, 0)),
        out_shape=jax.ShapeDtypeStruct((_NV, _W), jnp.float32),
    )(d0v, d1v, P, q0v, q1v, t2v, b2t, Bc)


def kernel(x, edge_index, W1, b1, W2, b2):
    src2 = edge_index[0]
    dst2 = edge_index[1]
    zeros_n = jnp.zeros((_NPAD,), jnp.float32)
    ones_ch = jnp.ones((_CH,), jnp.float32)

    eye = jnp.eye(_H, dtype=jnp.float32)
    P = jnp.kron(eye, jnp.ones((1, _H), jnp.float32))
    W2p = jnp.zeros((_H, _H), jnp.float32).at[:, : W2.shape[1]].set(W2)
    W2bd = jnp.kron(eye, W2p)
    b1t = jnp.tile(b1, _H).reshape(1, _W)
    b2p = jnp.zeros((_H,), jnp.float32).at[: b2.shape[0]].set(b2)
    b2t = jnp.tile(b2p, _H).reshape(1, _W)
    Bc = jnp.kron(eye, jnp.zeros((_H, _H), jnp.float32).at[0].set(1.0))

    d0p, d1p = _sc_degree(dst2, zeros_n, ones_ch)
    d0v = d0p.reshape(_NV, _H)
    d1v = d1p.reshape(_NV, _H)

    xt = x.T
    xt_tail = x[:, _KMAIN:].T
    h1 = _tc_mm(xt, xt_tail, W1[:_KMAIN], W1[_KMAIN:])
    t1v = _tc_scale(d0v, d1v, P, h1.reshape(_NV, _W))
    t1 = t1v.reshape(_NPAD, _H)
    p0, p1 = _sc_edge(t1, src2, dst2)

    t2v = _tc_layer2(d0v, d1v, P, p0.reshape(_NV, _W), p1.reshape(_NV, _W),
                     t1v, b1t, W2bd)
    t2 = t2v.reshape(_NPAD, _H)
    q0, q1 = _sc_edge(t2, src2, dst2)

    outv = _tc_final(d0v, d1v, P, q0.reshape(_NV, _W), q1.reshape(_NV, _W),
                     t2v, b2t, Bc)
    return outv.reshape(_NPAD, _H)[:_N, :7]


# speedup vs baseline: 1.7073x; 1.0326x over previous
import functools

import jax
import jax.numpy as jnp
from jax import lax
from jax.experimental import pallas as pl
from jax.experimental.pallas import tpu as pltpu
from jax.experimental.pallas import tpu_sc as plsc

_N = 10000
_E = 640000
_H = 16
_NC, _NS = 2, 16
_NW = _NC * _NS
_EPW = _E // _NW
_CH = 2000
_NCHUNK = _EPW // _CH
_NPAD = 10240
_RPT = _NPAD // _NS

_MESH = plsc.VectorSubcoreMesh(core_axis_name="c", subcore_axis_name="s")


@functools.partial(
    pl.kernel,
    out_type=(
        jax.ShapeDtypeStruct((_NPAD,), jnp.float32),
        jax.ShapeDtypeStruct((_NPAD,), jnp.float32),
    ),
    mesh=_MESH,
    scratch_types=[
        pltpu.VMEM((_CH,), jnp.int32),
        pltpu.VMEM((_CH,), jnp.float32),
        pltpu.VMEM_SHARED((_NPAD,), jnp.float32),
    ],
    compiler_params=pltpu.CompilerParams(use_tc_tiling_on_sc=False),
)
def _sc_degree(dst_hbm, zeros_hbm, ones_hbm, d0_hbm, d1_hbm, idx_v, ones_v, acc_sh):
    c = lax.axis_index("c")
    s = lax.axis_index("s")
    wid = s * _NC + c
    pltpu.sync_copy(zeros_hbm.at[pl.ds(s * _RPT, _RPT)], acc_sh.at[pl.ds(s * _RPT, _RPT)])
    pltpu.sync_copy(ones_hbm, ones_v)
    plsc.subcore_barrier()
    base = wid * _EPW
    for k in range(_NCHUNK):
        pltpu.sync_copy(dst_hbm.at[pl.ds(base + k * _CH, _CH)], idx_v)
        pltpu.sync_copy(ones_v, acc_sh.at[idx_v], add=True)
    plsc.subcore_barrier()

    @pl.when(c == 0)
    def _():
        pltpu.sync_copy(acc_sh.at[pl.ds(s * _RPT, _RPT)], d0_hbm.at[pl.ds(s * _RPT, _RPT)])

    @pl.when(c == 1)
    def _():
        pltpu.sync_copy(acc_sh.at[pl.ds(s * _RPT, _RPT)], d1_hbm.at[pl.ds(s * _RPT, _RPT)])


@functools.partial(
    pl.kernel,
    out_type=(
        jax.ShapeDtypeStruct((_NPAD, _H), jnp.float32),
        jax.ShapeDtypeStruct((_NPAD, _H), jnp.float32),
    ),
    mesh=_MESH,
    scratch_types=[
        pltpu.VMEM((_CH,), jnp.int32),
        pltpu.VMEM((_CH,), jnp.int32),
        pltpu.VMEM((_CH,), jnp.int32),
        pltpu.VMEM((_CH,), jnp.int32),
        pltpu.VMEM((_CH, _H), jnp.float32),
        pltpu.VMEM((_CH, _H), jnp.float32),
        pltpu.VMEM_SHARED((_NPAD, _H), jnp.float32),
        pltpu.SemaphoreType.DMA,
        pltpu.SemaphoreType.DMA,
        pltpu.SemaphoreType.DMA,
        pltpu.SemaphoreType.DMA,
    ],
    compiler_params=pltpu.CompilerParams(use_tc_tiling_on_sc=False),
)
def _sc_edge(t_hbm, src_hbm, dst_hbm, p0_hbm, p1_hbm,
             sidx0, sidx1, didx0, didx1, rows0, rows1, acc_sh,
             semg0, semg1, semi0, semi1):
    c = lax.axis_index("c")
    s = lax.axis_index("s")
    wid = s * _NC + c
    pltpu.sync_copy(t_hbm.at[pl.ds(s * _RPT, _RPT)], acc_sh.at[pl.ds(s * _RPT, _RPT)])
    plsc.subcore_barrier()
    base = wid * _EPW
    sidx = (sidx0, sidx1)
    didx = (didx0, didx1)
    rows = (rows0, rows1)
    semg = (semg0, semg1)
    semi = (semi0, semi1)

    def stage_idx(k):
        bb = k & 1
        off = base + k * _CH
        return (pltpu.async_copy(src_hbm.at[pl.ds(off, _CH)], sidx[bb], semi[bb]),
                pltpu.async_copy(dst_hbm.at[pl.ds(off, _CH)], didx[bb], semi[bb]))

    idesc = [None] * _NCHUNK
    gdesc = [None] * _NCHUNK
    idesc[0] = stage_idx(0)
    for d in idesc[0]:
        d.wait()
    gdesc[0] = pltpu.async_copy(t_hbm.at[sidx0], rows0, semg0)
    if _NCHUNK > 1:
        idesc[1] = stage_idx(1)
    for k in range(_NCHUNK):
        b = k & 1
        nb = (k + 1) & 1
        if k + 1 < _NCHUNK:
            for d in idesc[k + 1]:
                d.wait()
            gdesc[k + 1] = pltpu.async_copy(t_hbm.at[sidx[nb]], rows[nb], semg[nb])
        gdesc[k].wait()
        pltpu.sync_copy(rows[b], acc_sh.at[didx[b]], add=True)
        if k + 2 < _NCHUNK:
            idesc[k + 2] = stage_idx(k + 2)
    plsc.subcore_barrier()

    @pl.when(c == 0)
    def _():
        pltpu.sync_copy(acc_sh.at[pl.ds(s * _RPT, _RPT)], p0_hbm.at[pl.ds(s * _RPT, _RPT)])

    @pl.when(c == 1)
    def _():
        pltpu.sync_copy(acc_sh.at[pl.ds(s * _RPT, _RPT)], p1_hbm.at[pl.ds(s * _RPT, _RPT)])


_BLKP = 1024


_F_IN = 1433
_KMAIN = 1408
_KCH = 352
_KTAIL = _F_IN - _KMAIN
_NKCH = _KMAIN // _KCH


def _mm_body(xt_ref, xtail_ref, wm_ref, wt_ref, o_ref):
    i = pl.program_id(0)
    part = lax.dot_general(
        xt_ref[...], wm_ref[...], (((0,), (0,)), ((), ())),
        preferred_element_type=jnp.float32)

    @pl.when(i == 0)
    def _():
        o_ref[pl.ds(0, _N), :] = part

    @pl.when(i > 0)
    def _():
        o_ref[pl.ds(0, _N), :] += part

    @pl.when(i == _NKCH - 1)
    def _():
        tail = lax.dot_general(
            xtail_ref[...], wt_ref[...], (((0,), (0,)), ((), ())),
            preferred_element_type=jnp.float32)
        o_ref[pl.ds(0, _N), :] += tail
        o_ref[pl.ds(_N, _NPAD - _N), :] = jnp.zeros((_NPAD - _N, _H), jnp.float32)


def _tc_mm(xt, xt_tail, W1m, W1t):
    return pl.pallas_call(
        _mm_body,
        grid=(_NKCH,),
        in_specs=[
            pl.BlockSpec((_KCH, _N), lambda i: (i, 0)),
            pl.BlockSpec((_KTAIL, _N), lambda i: (0, 0)),
            pl.BlockSpec((_KCH, _H), lambda i: (i, 0)),
            pl.BlockSpec((_KTAIL, _H), lambda i: (0, 0)),
        ],
        out_specs=pl.BlockSpec((_NPAD, _H), lambda i: (0, 0)),
        out_shape=jax.ShapeDtypeStruct((_NPAD, _H), jnp.float32),
        compiler_params=pltpu.CompilerParams(fuse_transposed_lhs_in_matmul=True),
    )(xt, xt_tail, W1m, W1t)


_NV = _NPAD // _H
_W = _H * _H


def _scale_body(d0_ref, d1_ref, p_ref, h_ref, o_ref):
    dinv16 = lax.rsqrt(d0_ref[...] + d1_ref[...] + 1.0)
    dinvb = jnp.dot(dinv16, p_ref[...], preferred_element_type=jnp.float32)
    o_ref[...] = h_ref[...] * dinvb


def _tc_scale(d0v, d1v, P, hv):
    return pl.pallas_call(
        _scale_body,
        grid=(1,),
        in_specs=[
            pl.BlockSpec((_NV, _H), lambda i: (0, 0)),
            pl.BlockSpec((_NV, _H), lambda i: (0, 0)),
            pl.BlockSpec((_H, _W), lambda i: (0, 0)),
            pl.BlockSpec((_NV, _W), lambda i: (0, 0)),
        ],
        out_specs=pl.BlockSpec((_NV, _W), lambda i: (0, 0)),
        out_shape=jax.ShapeDtypeStruct((_NV, _W), jnp.float32),
    )(d0v, d1v, P, hv)


def _layer2_body(d0_ref, d1_ref, p_ref, p0_ref, p1_ref, t1_ref, b1_ref, w2_ref, o_ref):
    dinv16 = lax.rsqrt(d0_ref[...] + d1_ref[...] + 1.0)
    dinvb = jnp.dot(dinv16, p_ref[...], preferred_element_type=jnp.float32)
    agg = p0_ref[...] + p1_ref[...] - t1_ref[...]
    h1 = jnp.maximum(dinvb * agg + b1_ref[...], 0.0)
    h2 = jnp.dot(h1, w2_ref[...], preferred_element_type=jnp.float32)
    o_ref[...] = dinvb * h2


def _tc_layer2(d0v, d1v, P, p0v, p1v, t1v, b1t, W2bd):
    return pl.pallas_call(
        _layer2_body,
        grid=(1,),
        in_specs=[
            pl.BlockSpec((_NV, _H), lambda i: (0, 0)),
            pl.BlockSpec((_NV, _H), lambda i: (0, 0)),
            pl.BlockSpec((_H, _W), lambda i: (0, 0)),
            pl.BlockSpec((_NV, _W), lambda i: (0, 0)),
            pl.BlockSpec((_NV, _W), lambda i: (0, 0)),
            pl.BlockSpec((_NV, _W), lambda i: (0, 0)),
            pl.BlockSpec((1, _W), lambda i: (0, 0)),
            pl.BlockSpec((_W, _W), lambda i: (0, 0)),
        ],
        out_specs=pl.BlockSpec((_NV, _W), lambda i: (0, 0)),
        out_shape=jax.ShapeDtypeStruct((_NV, _W), jnp.float32),
    )(d0v, d1v, P, p0v, p1v, t1v, b1t, W2bd)


def _final_body(d0_ref, d1_ref, p_ref, q0_ref, q1_ref, t2_ref, b2_ref, bc_ref, o_ref):
    dinv16 = lax.rsqrt(d0_ref[...] + d1_ref[...] + 1.0)
    dinvb = jnp.dot(dinv16, p_ref[...], preferred_element_type=jnp.float32)
    z = dinvb * (q0_ref[...] + q1_ref[...] - t2_ref[...]) + b2_ref[...]
    col = lax.broadcasted_iota(jnp.int32, z.shape, 1)
    cls = col & (_H - 1)
    valid = cls < 7
    gs = cls == 0
    zm = jnp.where(valid, z, -1e30)
    m = jnp.maximum(zm, pltpu.roll(zm, _W - 1, 1))
    m = jnp.maximum(m, pltpu.roll(m, _W - 2, 1))
    m = jnp.maximum(m, pltpu.roll(m, _W - 4, 1))
    mB = jnp.dot(jnp.where(gs, m, 0.0), bc_ref[...], preferred_element_type=jnp.float32)
    e = jnp.where(valid, jnp.exp(z - mB), 0.0)
    s = e + pltpu.roll(e, _W - 1, 1)
    s = s + pltpu.roll(s, _W - 2, 1)
    s = s + pltpu.roll(s, _W - 4, 1)
    sB = jnp.dot(jnp.where(gs, s, 0.0), bc_ref[...], preferred_element_type=jnp.float32)
    o_ref[...] = z - mB - jnp.log(sB)


def _tc_final(d0v, d1v, P, q0v, q1v, t2v, b2t, Bc):
    return pl.pallas_call(
        _final_body,
        grid=(1,),
        in_specs=[
            pl.BlockSpec((_NV, _H), lambda i: (0, 0)),
            pl.BlockSpec((_NV, _H), lambda i: (0, 0)),
            pl.BlockSpec((_H, _W), lambda i: (0, 0)),
            pl.BlockSpec((_NV, _W), lambda i: (0, 0)),
            pl.BlockSpec((_NV, _W), lambda i: (0, 0)),
            pl.BlockSpec((_NV, _W), lambda i: (0, 0)),
            pl.BlockSpec((1, _W), lambda i: (0, 0)),
            pl.BlockSpec((_W, _W), lambda i: (0, 0)),
        ],
        out_specs=pl.BlockSpec((_NV, _W), lambda i: (0, 0)),
        out_shape=jax.ShapeDtypeStruct((_NV, _W), jnp.float32),
    )(d0v, d1v, P, q0v, q1v, t2v, b2t, Bc)


def kernel(x, edge_index, W1, b1, W2, b2):
    src2 = edge_index[0]
    dst2 = edge_index[1]
    zeros_n = jnp.zeros((_NPAD,), jnp.float32)
    ones_ch = jnp.ones((_CH,), jnp.float32)

    eye = jnp.eye(_H, dtype=jnp.float32)
    P = jnp.kron(eye, jnp.ones((1, _H), jnp.float32))
    W2p = jnp.zeros((_H, _H), jnp.float32).at[:, : W2.shape[1]].set(W2)
    W2bd = jnp.kron(eye, W2p)
    b1t = jnp.tile(b1, _H).reshape(1, _W)
    b2p = jnp.zeros((_H,), jnp.float32).at[: b2.shape[0]].set(b2)
    b2t = jnp.tile(b2p, _H).reshape(1, _W)
    Bc = jnp.kron(eye, jnp.zeros((_H, _H), jnp.float32).at[0].set(1.0))

    d0p, d1p = _sc_degree(dst2, zeros_n, ones_ch)
    d0v = d0p.reshape(_NV, _H)
    d1v = d1p.reshape(_NV, _H)

    xt = x.T
    xt_tail = x[:, _KMAIN:].T
    h1 = _tc_mm(xt, xt_tail, W1[:_KMAIN], W1[_KMAIN:])
    t1v = _tc_scale(d0v, d1v, P, h1.reshape(_NV, _W))
    t1 = t1v.reshape(_NPAD, _H)
    p0, p1 = _sc_edge(t1, src2, dst2)

    t2v = _tc_layer2(d0v, d1v, P, p0.reshape(_NV, _W), p1.reshape(_NV, _W),
                     t1v, b1t, W2bd)
    t2 = t2v.reshape(_NPAD, _H)
    q0, q1 = _sc_edge(t2, src2, dst2)

    outv = _tc_final(d0v, d1v, P, q0.reshape(_NV, _W), q1.reshape(_NV, _W),
                     t2v, b2t, Bc)
    return outv.reshape(_NPAD, _H)[:_N, :7]


# speedup vs baseline: 1.7161x; 1.0052x over previous
import functools

import jax
import jax.numpy as jnp
from jax import lax
from jax.experimental import pallas as pl
from jax.experimental.pallas import tpu as pltpu
from jax.experimental.pallas import tpu_sc as plsc

_N = 10000
_E = 640000
_H = 16
_NC, _NS = 2, 16
_NW = _NC * _NS
_EPW = _E // _NW
_CH = 2000
_NCHUNK = _EPW // _CH
_NPAD = 10240
_RPT = _NPAD // _NS

_MESH = plsc.VectorSubcoreMesh(core_axis_name="c", subcore_axis_name="s")


@functools.partial(
    pl.kernel,
    out_type=(
        jax.ShapeDtypeStruct((_NPAD,), jnp.float32),
        jax.ShapeDtypeStruct((_NPAD,), jnp.float32),
    ),
    mesh=_MESH,
    scratch_types=[
        pltpu.VMEM((_CH,), jnp.int32),
        pltpu.VMEM((_CH,), jnp.float32),
        pltpu.VMEM_SHARED((_NPAD,), jnp.float32),
    ],
    compiler_params=pltpu.CompilerParams(use_tc_tiling_on_sc=False),
)
def _sc_degree(dst_hbm, zeros_hbm, ones_hbm, d0_hbm, d1_hbm, idx_v, ones_v, acc_sh):
    c = lax.axis_index("c")
    s = lax.axis_index("s")
    wid = s * _NC + c
    pltpu.sync_copy(zeros_hbm.at[pl.ds(s * _RPT, _RPT)], acc_sh.at[pl.ds(s * _RPT, _RPT)])
    pltpu.sync_copy(ones_hbm, ones_v)
    plsc.subcore_barrier()
    base = wid * _EPW
    for k in range(_NCHUNK):
        pltpu.sync_copy(dst_hbm.at[pl.ds(base + k * _CH, _CH)], idx_v)
        pltpu.sync_copy(ones_v, acc_sh.at[idx_v], add=True)
    plsc.subcore_barrier()

    @pl.when(c == 0)
    def _():
        pltpu.sync_copy(acc_sh.at[pl.ds(s * _RPT, _RPT)], d0_hbm.at[pl.ds(s * _RPT, _RPT)])

    @pl.when(c == 1)
    def _():
        pltpu.sync_copy(acc_sh.at[pl.ds(s * _RPT, _RPT)], d1_hbm.at[pl.ds(s * _RPT, _RPT)])


@functools.partial(
    pl.kernel,
    out_type=(
        jax.ShapeDtypeStruct((_NPAD, _H), jnp.float32),
        jax.ShapeDtypeStruct((_NPAD, _H), jnp.float32),
    ),
    mesh=_MESH,
    scratch_types=[
        pltpu.VMEM((_CH,), jnp.int32),
        pltpu.VMEM((_CH,), jnp.int32),
        pltpu.VMEM((_CH,), jnp.int32),
        pltpu.VMEM((_CH,), jnp.int32),
        pltpu.VMEM((_CH,), jnp.int32),
        pltpu.VMEM((_CH,), jnp.int32),
        pltpu.VMEM((_CH, _H), jnp.float32),
        pltpu.VMEM((_CH, _H), jnp.float32),
        pltpu.VMEM((_CH, _H), jnp.float32),
        pltpu.VMEM_SHARED((_NPAD, _H), jnp.float32),
        pltpu.SemaphoreType.DMA,
        pltpu.SemaphoreType.DMA,
        pltpu.SemaphoreType.DMA,
        pltpu.SemaphoreType.DMA,
        pltpu.SemaphoreType.DMA,
        pltpu.SemaphoreType.DMA,
        pltpu.SemaphoreType.DMA,
        pltpu.SemaphoreType.DMA,
        pltpu.SemaphoreType.DMA,
    ],
    compiler_params=pltpu.CompilerParams(use_tc_tiling_on_sc=False),
)
def _sc_edge(t_hbm, src_hbm, dst_hbm, p0_hbm, p1_hbm,
             sidx0, sidx1, sidx2, didx0, didx1, didx2, rows0, rows1, rows2, acc_sh,
             semg0, semg1, semg2, semi0, semi1, semi2, sems0, sems1, sems2):
    c = lax.axis_index("c")
    s = lax.axis_index("s")
    wid = s * _NC + c
    pltpu.sync_copy(t_hbm.at[pl.ds(s * _RPT, _RPT)], acc_sh.at[pl.ds(s * _RPT, _RPT)])
    plsc.subcore_barrier()
    base = wid * _EPW
    sidx = (sidx0, sidx1, sidx2)
    didx = (didx0, didx1, didx2)
    rows = (rows0, rows1, rows2)
    semg = (semg0, semg1, semg2)
    semi = (semi0, semi1, semi2)
    sems = (sems0, sems1, sems2)

    def stage_idx(k):
        bb = k % 3
        off = base + k * _CH
        return (pltpu.async_copy(src_hbm.at[pl.ds(off, _CH)], sidx[bb], semi[bb]),
                pltpu.async_copy(dst_hbm.at[pl.ds(off, _CH)], didx[bb], semi[bb]))

    idesc = [None] * _NCHUNK
    gdesc = [None] * _NCHUNK
    sdesc = [None] * _NCHUNK
    swaited = set()

    def wait_scatter(k):
        if 0 <= k < _NCHUNK and k not in swaited:
            swaited.add(k)
            sdesc[k].wait()

    idesc[0] = stage_idx(0)
    for d in idesc[0]:
        d.wait()
    gdesc[0] = pltpu.async_copy(t_hbm.at[sidx0], rows0, semg0)
    if _NCHUNK > 1:
        idesc[1] = stage_idx(1)
    for k in range(_NCHUNK):
        b = k % 3
        nb = (k + 1) % 3
        if k + 1 < _NCHUNK:
            for d in idesc[k + 1]:
                d.wait()
            wait_scatter(k - 2)
            gdesc[k + 1] = pltpu.async_copy(t_hbm.at[sidx[nb]], rows[nb], semg[nb])
        gdesc[k].wait()
        sdesc[k] = pltpu.async_copy(rows[b], acc_sh.at[didx[b]], sems[b], add=True)
        if k + 2 < _NCHUNK:
            wait_scatter(k - 1)
            idesc[k + 2] = stage_idx(k + 2)
    for k in range(_NCHUNK):
        wait_scatter(k)
    plsc.subcore_barrier()

    @pl.when(c == 0)
    def _():
        pltpu.sync_copy(acc_sh.at[pl.ds(s * _RPT, _RPT)], p0_hbm.at[pl.ds(s * _RPT, _RPT)])

    @pl.when(c == 1)
    def _():
        pltpu.sync_copy(acc_sh.at[pl.ds(s * _RPT, _RPT)], p1_hbm.at[pl.ds(s * _RPT, _RPT)])


_BLKP = 1024


_F_IN = 1433
_KMAIN = 1408
_KCH = 352
_KTAIL = _F_IN - _KMAIN
_NKCH = _KMAIN // _KCH


def _mm_body(xt_ref, xtail_ref, wm_ref, wt_ref, o_ref):
    i = pl.program_id(0)
    part = lax.dot_general(
        xt_ref[...], wm_ref[...], (((0,), (0,)), ((), ())),
        preferred_element_type=jnp.float32)

    @pl.when(i == 0)
    def _():
        o_ref[pl.ds(0, _N), :] = part

    @pl.when(i > 0)
    def _():
        o_ref[pl.ds(0, _N), :] += part

    @pl.when(i == _NKCH - 1)
    def _():
        tail = lax.dot_general(
            xtail_ref[...], wt_ref[...], (((0,), (0,)), ((), ())),
            preferred_element_type=jnp.float32)
        o_ref[pl.ds(0, _N), :] += tail
        o_ref[pl.ds(_N, _NPAD - _N), :] = jnp.zeros((_NPAD - _N, _H), jnp.float32)


def _tc_mm(xt, xt_tail, W1m, W1t):
    return pl.pallas_call(
        _mm_body,
        grid=(_NKCH,),
        in_specs=[
            pl.BlockSpec((_KCH, _N), lambda i: (i, 0)),
            pl.BlockSpec((_KTAIL, _N), lambda i: (0, 0)),
            pl.BlockSpec((_KCH, _H), lambda i: (i, 0)),
            pl.BlockSpec((_KTAIL, _H), lambda i: (0, 0)),
        ],
        out_specs=pl.BlockSpec((_NPAD, _H), lambda i: (0, 0)),
        out_shape=jax.ShapeDtypeStruct((_NPAD, _H), jnp.float32),
        compiler_params=pltpu.CompilerParams(fuse_transposed_lhs_in_matmul=True),
    )(xt, xt_tail, W1m, W1t)


_NV = _NPAD // _H
_W = _H * _H


def _scale_body(d0_ref, d1_ref, p_ref, h_ref, o_ref):
    dinv16 = lax.rsqrt(d0_ref[...] + d1_ref[...] + 1.0)
    dinvb = jnp.dot(dinv16, p_ref[...], preferred_element_type=jnp.float32)
    o_ref[...] = h_ref[...] * dinvb


def _tc_scale(d0v, d1v, P, hv):
    return pl.pallas_call(
        _scale_body,
        grid=(1,),
        in_specs=[
            pl.BlockSpec((_NV, _H), lambda i: (0, 0)),
            pl.BlockSpec((_NV, _H), lambda i: (0, 0)),
            pl.BlockSpec((_H, _W), lambda i: (0, 0)),
            pl.BlockSpec((_NV, _W), lambda i: (0, 0)),
        ],
        out_specs=pl.BlockSpec((_NV, _W), lambda i: (0, 0)),
        out_shape=jax.ShapeDtypeStruct((_NV, _W), jnp.float32),
    )(d0v, d1v, P, hv)


def _layer2_body(d0_ref, d1_ref, p_ref, p0_ref, p1_ref, t1_ref, b1_ref, w2_ref, o_ref):
    dinv16 = lax.rsqrt(d0_ref[...] + d1_ref[...] + 1.0)
    dinvb = jnp.dot(dinv16, p_ref[...], preferred_element_type=jnp.float32)
    agg = p0_ref[...] + p1_ref[...] - t1_ref[...]
    h1 = jnp.maximum(dinvb * agg + b1_ref[...], 0.0)
    h2 = jnp.dot(h1, w2_ref[...], preferred_element_type=jnp.float32)
    o_ref[...] = dinvb * h2


def _tc_layer2(d0v, d1v, P, p0v, p1v, t1v, b1t, W2bd):
    return pl.pallas_call(
        _layer2_body,
        grid=(1,),
        in_specs=[
            pl.BlockSpec((_NV, _H), lambda i: (0, 0)),
            pl.BlockSpec((_NV, _H), lambda i: (0, 0)),
            pl.BlockSpec((_H, _W), lambda i: (0, 0)),
            pl.BlockSpec((_NV, _W), lambda i: (0, 0)),
            pl.BlockSpec((_NV, _W), lambda i: (0, 0)),
            pl.BlockSpec((_NV, _W), lambda i: (0, 0)),
            pl.BlockSpec((1, _W), lambda i: (0, 0)),
            pl.BlockSpec((_W, _W), lambda i: (0, 0)),
        ],
        out_specs=pl.BlockSpec((_NV, _W), lambda i: (0, 0)),
        out_shape=jax.ShapeDtypeStruct((_NV, _W), jnp.float32),
    )(d0v, d1v, P, p0v, p1v, t1v, b1t, W2bd)


def _final_body(d0_ref, d1_ref, p_ref, q0_ref, q1_ref, t2_ref, b2_ref, bc_ref, o_ref):
    dinv16 = lax.rsqrt(d0_ref[...] + d1_ref[...] + 1.0)
    dinvb = jnp.dot(dinv16, p_ref[...], preferred_element_type=jnp.float32)
    z = dinvb * (q0_ref[...] + q1_ref[...] - t2_ref[...]) + b2_ref[...]
    col = lax.broadcasted_iota(jnp.int32, z.shape, 1)
    cls = col & (_H - 1)
    valid = cls < 7
    gs = cls == 0
    zm = jnp.where(valid, z, -1e30)
    m = jnp.maximum(zm, pltpu.roll(zm, _W - 1, 1))
    m = jnp.maximum(m, pltpu.roll(m, _W - 2, 1))
    m = jnp.maximum(m, pltpu.roll(m, _W - 4, 1))
    mB = jnp.dot(jnp.where(gs, m, 0.0), bc_ref[...], preferred_element_type=jnp.float32)
    e = jnp.where(valid, jnp.exp(z - mB), 0.0)
    s = e + pltpu.roll(e, _W - 1, 1)
    s = s + pltpu.roll(s, _W - 2, 1)
    s = s + pltpu.roll(s, _W - 4, 1)
    sB = jnp.dot(jnp.where(gs, s, 0.0), bc_ref[...], preferred_element_type=jnp.float32)
    o_ref[...] = z - mB - jnp.log(sB)


def _tc_final(d0v, d1v, P, q0v, q1v, t2v, b2t, Bc):
    return pl.pallas_call(
        _final_body,
        grid=(1,),
        in_specs=[
            pl.BlockSpec((_NV, _H), lambda i: (0, 0)),
            pl.BlockSpec((_NV, _H), lambda i: (0, 0)),
            pl.BlockSpec((_H, _W), lambda i: (0, 0)),
            pl.BlockSpec((_NV, _W), lambda i: (0, 0)),
            pl.BlockSpec((_NV, _W), lambda i: (0, 0)),
            pl.BlockSpec((_NV, _W), lambda i: (0, 0)),
            pl.BlockSpec((1, _W), lambda i: (0, 0)),
            pl.BlockSpec((_W, _W), lambda i: (0, 0)),
        ],
        out_specs=pl.BlockSpec((_NV, _W), lambda i: (0, 0)),
        out_shape=jax.ShapeDtypeStruct((_NV, _W), jnp.float32),
    )(d0v, d1v, P, q0v, q1v, t2v, b2t, Bc)


def kernel(x, edge_index, W1, b1, W2, b2):
    src2 = edge_index[0]
    dst2 = edge_index[1]
    zeros_n = jnp.zeros((_NPAD,), jnp.float32)
    ones_ch = jnp.ones((_CH,), jnp.float32)

    eye = jnp.eye(_H, dtype=jnp.float32)
    P = jnp.kron(eye, jnp.ones((1, _H), jnp.float32))
    W2p = jnp.zeros((_H, _H), jnp.float32).at[:, : W2.shape[1]].set(W2)
    W2bd = jnp.kron(eye, W2p)
    b1t = jnp.tile(b1, _H).reshape(1, _W)
    b2p = jnp.zeros((_H,), jnp.float32).at[: b2.shape[0]].set(b2)
    b2t = jnp.tile(b2p, _H).reshape(1, _W)
    Bc = jnp.kron(eye, jnp.zeros((_H, _H), jnp.float32).at[0].set(1.0))

    d0p, d1p = _sc_degree(dst2, zeros_n, ones_ch)
    d0v = d0p.reshape(_NV, _H)
    d1v = d1p.reshape(_NV, _H)

    xt = x.T
    xt_tail = x[:, _KMAIN:].T
    h1 = _tc_mm(xt, xt_tail, W1[:_KMAIN], W1[_KMAIN:])
    t1v = _tc_scale(d0v, d1v, P, h1.reshape(_NV, _W))
    t1 = t1v.reshape(_NPAD, _H)
    p0, p1 = _sc_edge(t1, src2, dst2)

    t2v = _tc_layer2(d0v, d1v, P, p0.reshape(_NV, _W), p1.reshape(_NV, _W),
                     t1v, b1t, W2bd)
    t2 = t2v.reshape(_NPAD, _H)
    q0, q1 = _sc_edge(t2, src2, dst2)

    outv = _tc_final(d0v, d1v, P, q0.reshape(_NV, _W), q1.reshape(_NV, _W),
                     t2v, b2t, Bc)
    return outv.reshape(_NPAD, _H)[:_N, :7]
